# baseline (speedup 1.0000x reference)
import functools

import jax
import jax.numpy as jnp
from jax import lax
from jax.experimental import pallas as pl
from jax.experimental.pallas import tpu as pltpu
from jax.experimental.pallas import tpu_sc as plsc

_NC = 2
_NS = 16
_NW = _NC * _NS
_LANES = 16


def _sc_gather_stats(output, idx3, weights, *, B, C, bpw, K, nchunk):
    cf = C // _LANES
    ct = C - cf * _LANES

    mesh = plsc.VectorSubcoreMesh(core_axis_name="c", subcore_axis_name="s")

    @functools.partial(
        pl.kernel,
        mesh=mesh,
        compiler_params=pltpu.CompilerParams(use_tc_tiling_on_sc=False),
        out_type=[
            jax.ShapeDtypeStruct((B, _LANES), jnp.float32),
            jax.ShapeDtypeStruct((_NW, _LANES), jnp.float32),
        ],
        scratch_types=[
            pltpu.VMEM((nchunk, K), jnp.int32),
            pltpu.VMEM((K, C), jnp.float32),
            pltpu.VMEM((K, C), jnp.float32),
            pltpu.VMEM((bpw, _LANES), jnp.float32),
            pltpu.VMEM((_LANES,), jnp.float32),
        ],
    )
    def k(out_hbm, idx_hbm, w_hbm, g2_hbm, t1_hbm, idx_v, w_v, o_v, g2_v, t1_v):
        cid = lax.axis_index("c")
        sid = lax.axis_index("s")
        wid = sid * _NC + cid
        base = wid * bpw

        pltpu.sync_copy(idx_hbm.at[wid], idx_v)

        if ct:
            tailmask = jnp.where(lax.iota(jnp.int32, _LANES) < (_LANES - ct),
                                 0.0, 1.0).astype(jnp.float32)

        acc1 = jnp.zeros((_LANES,), jnp.float32)
        for ch in range(nchunk):
            pltpu.sync_copy(w_hbm.at[idx_v.at[ch]], w_v)
            pltpu.sync_copy(out_hbm.at[pl.ds(base + ch * K, K)], o_v)

            def row_body(r, a1):
                def col_body(j, carry):
                    c1, c2 = carry
                    off = pl.multiple_of(j * _LANES, _LANES)
                    wv = w_v[r, pl.ds(off, _LANES)]
                    ov = o_v[r, pl.ds(off, _LANES)]
                    return c1 + wv * ov, c2 + wv

                a1, a2 = lax.fori_loop(
                    0, cf, col_body, (a1, jnp.zeros((_LANES,), jnp.float32)))
                if ct:
                    wv = w_v[r, pl.ds(C - _LANES, _LANES)] * tailmask
                    ov = o_v[r, pl.ds(C - _LANES, _LANES)]
                    a1 = a1 + wv * ov
                    a2 = a2 + wv
                g2_v[ch * K + r, :] = a2
                return a1

            acc1 = lax.fori_loop(0, K, row_body, acc1)

        t1_v[:] = acc1
        pltpu.sync_copy(g2_v, g2_hbm.at[pl.ds(base, bpw)])
        pltpu.sync_copy(t1_v, t1_hbm.at[wid])

    return k(output, idx3, weights)


def _tc_combine(output, g2part, t1part, *, B, C):
    BLK = 256
    grid = (B // BLK,)

    def body(out_ref, g2_ref, t1_ref, L_ref):
        j = pl.program_id(0)
        x = out_ref[...]
        m = jnp.max(x, axis=1, keepdims=True)
        lse = m + jnp.log(jnp.sum(jnp.exp(x - m), axis=1, keepdims=True))
        g2 = jnp.sum(g2_ref[...], axis=1, keepdims=True)

        @pl.when(j == 0)
        def _():
            L_ref[...] = -jnp.sum(t1_ref[...], keepdims=True).reshape(1, 1) / B

        L_ref[...] += jnp.sum(lse * g2, keepdims=True).reshape(1, 1) / B

    L = pl.pallas_call(
        body,
        grid=grid,
        in_specs=[
            pl.BlockSpec((BLK, C), lambda j: (j, 0)),
            pl.BlockSpec((BLK, _LANES), lambda j: (j, 0)),
            pl.BlockSpec((_NW, _LANES), lambda j: (0, 0)),
        ],
        out_specs=pl.BlockSpec((1, 1), lambda j: (0, 0)),
        out_shape=jax.ShapeDtypeStruct((1, 1), jnp.float32),
    )(output, g2part, t1part)
    return L[0, 0]


def kernel(output, targets, indices, weights):
    B, C = output.shape
    bpw = B // _NW
    K = 16
    nchunk = bpw // K
    idx3 = indices.reshape(_NW, nchunk, K)
    g2part, t1part = _sc_gather_stats(
        output, idx3, weights, B=B, C=C, bpw=bpw, K=K, nchunk=nchunk)
    return _tc_combine(output, g2part, t1part, B=B, C=C)


# speedup vs baseline: 1.1076x; 1.1076x over previous
import functools

import jax
import jax.numpy as jnp
from jax import lax
from jax.experimental import pallas as pl
from jax.experimental.pallas import tpu as pltpu
from jax.experimental.pallas import tpu_sc as plsc

_NC = 2
_NS = 16
_NW = _NC * _NS
_LANES = 16


def _sc_gather_stats(output_p, idx3, weights_p, *, B, CP, bpw, K, nchunk):
    cf = CP // _LANES

    mesh = plsc.VectorSubcoreMesh(core_axis_name="c", subcore_axis_name="s")

    @functools.partial(
        pl.kernel,
        mesh=mesh,
        out_type=[
            jax.ShapeDtypeStruct((B, _LANES), jnp.float32),
            jax.ShapeDtypeStruct((_NW, _LANES), jnp.float32),
        ],
        scratch_types=[
            pltpu.VMEM((nchunk, K), jnp.int32),
            pltpu.VMEM((K, CP), jnp.float32),
            pltpu.VMEM((K, CP), jnp.float32),
            pltpu.VMEM((bpw, _LANES), jnp.float32),
            pltpu.VMEM((_LANES,), jnp.float32),
        ],
    )
    def k(out_hbm, idx_hbm, w_hbm, g2_hbm, t1_hbm, idx_v, w_v, o_v, g2_v, t1_v):
        cid = lax.axis_index("c")
        sid = lax.axis_index("s")
        wid = sid * _NC + cid
        base = wid * bpw

        pltpu.sync_copy(idx_hbm.at[wid], idx_v)

        acc1 = jnp.zeros((_LANES,), jnp.float32)
        for ch in range(nchunk):
            pltpu.sync_copy(w_hbm.at[idx_v.at[ch]], w_v)
            pltpu.sync_copy(out_hbm.at[pl.ds(base + ch * K, K)], o_v)

            def row_body(r, a1):
                def col_body(j, carry):
                    c1, c2 = carry
                    off = pl.multiple_of(j * _LANES, _LANES)
                    wv = w_v[r, pl.ds(off, _LANES)]
                    ov = o_v[r, pl.ds(off, _LANES)]
                    return c1 + wv * ov, c2 + wv

                a1, a2 = lax.fori_loop(
                    0, cf, col_body, (a1, jnp.zeros((_LANES,), jnp.float32)))
                g2_v[ch * K + r, :] = a2
                return a1

            acc1 = lax.fori_loop(0, K, row_body, acc1)

        t1_v[:] = acc1
        pltpu.sync_copy(g2_v, g2_hbm.at[pl.ds(base, bpw)])
        pltpu.sync_copy(t1_v, t1_hbm.at[wid])

    return k(output_p, idx3, weights_p)


def _tc_combine(output, g2part, t1part, *, B, C):
    BLK = 256
    grid = (B // BLK,)

    def body(out_ref, g2_ref, t1_ref, L_ref):
        j = pl.program_id(0)
        x = out_ref[...]
        m = jnp.max(x, axis=1, keepdims=True)
        lse = m + jnp.log(jnp.sum(jnp.exp(x - m), axis=1, keepdims=True))
        g2 = jnp.sum(g2_ref[...], axis=1, keepdims=True)

        @pl.when(j == 0)
        def _():
            L_ref[...] = -jnp.sum(t1_ref[...], keepdims=True).reshape(1, 1) / B

        L_ref[...] += jnp.sum(lse * g2, keepdims=True).reshape(1, 1) / B

    L = pl.pallas_call(
        body,
        grid=grid,
        in_specs=[
            pl.BlockSpec((BLK, C), lambda j: (j, 0)),
            pl.BlockSpec((BLK, _LANES), lambda j: (j, 0)),
            pl.BlockSpec((_NW, _LANES), lambda j: (0, 0)),
        ],
        out_specs=pl.BlockSpec((1, 1), lambda j: (0, 0)),
        out_shape=jax.ShapeDtypeStruct((1, 1), jnp.float32),
    )(output, g2part, t1part)
    return L[0, 0]


def kernel(output, targets, indices, weights):
    B, C = output.shape
    CP = (C + 127) // 128 * 128
    if CP != C:
        weights_p = jnp.pad(weights, ((0, 0), (0, CP - C)))
        output_p = jnp.pad(output, ((0, 0), (0, CP - C)))
    else:
        weights_p, output_p = weights, output
    bpw = B // _NW
    K = 16
    nchunk = bpw // K
    idx3 = indices.reshape(_NW, nchunk, K)
    g2part, t1part = _sc_gather_stats(
        output_p, idx3, weights_p, B=B, CP=CP, bpw=bpw, K=K, nchunk=nchunk)
    return _tc_combine(output, g2part, t1part, B=B, C=C)


# speedup vs baseline: 2.7043x; 2.4415x over previous
import functools

import jax
import jax.numpy as jnp
from jax import lax
from jax.experimental import pallas as pl
from jax.experimental.pallas import tpu as pltpu
from jax.experimental.pallas import tpu_sc as plsc

_NC = 2
_NS = 16
_NW = _NC * _NS
_LANES = 16


def _tc_pad_rows(weights, *, CP):
    N, C = weights.shape
    BLK = 400
    grid = (N // BLK,)

    def body(w_ref, o_ref):
        o_ref[...] = jnp.concatenate(
            [w_ref[...], jnp.zeros((BLK, CP - C), jnp.float32)], axis=1)

    return pl.pallas_call(
        body,
        grid=grid,
        in_specs=[pl.BlockSpec((BLK, C), lambda j: (j, 0))],
        out_specs=pl.BlockSpec((BLK, CP), lambda j: (j, 0)),
        out_shape=jax.ShapeDtypeStruct((N, CP), jnp.float32),
    )(weights)


def _sc_gather_stats(output, idx3, weights_p, *, B, C, bpw, K, nchunk):
    cf = C // _LANES
    ct = C - cf * _LANES
    CP = weights_p.shape[1]

    mesh = plsc.VectorSubcoreMesh(core_axis_name="c", subcore_axis_name="s")

    @functools.partial(
        pl.kernel,
        mesh=mesh,
        out_type=[
            jax.ShapeDtypeStruct((B, _LANES), jnp.float32),
            jax.ShapeDtypeStruct((_NW, _LANES), jnp.float32),
        ],
        scratch_types=[
            pltpu.VMEM((nchunk, K), jnp.int32),
            pltpu.VMEM((K, CP), jnp.float32),
            pltpu.VMEM((K, C), jnp.float32),
            pltpu.VMEM((bpw, _LANES), jnp.float32),
            pltpu.VMEM((_LANES,), jnp.float32),
        ],
    )
    def k(out_hbm, idx_hbm, w_hbm, g2_hbm, t1_hbm, idx_v, w_v, o_v, g2_v, t1_v):
        cid = lax.axis_index("c")
        sid = lax.axis_index("s")
        wid = sid * _NC + cid
        base = wid * bpw

        pltpu.sync_copy(idx_hbm.at[wid], idx_v)

        if ct:
            tailmask = jnp.where(lax.iota(jnp.int32, _LANES) < (_LANES - ct),
                                 0.0, 1.0).astype(jnp.float32)

        acc1 = jnp.zeros((_LANES,), jnp.float32)
        for ch in range(nchunk):
            pltpu.sync_copy(w_hbm.at[idx_v.at[ch]], w_v)
            pltpu.sync_copy(out_hbm.at[pl.ds(base + ch * K, K)], o_v)

            def row_body(r, a1):
                def col_body(j, carry):
                    c1, c2 = carry
                    off = pl.multiple_of(j * _LANES, _LANES)
                    wv = w_v[r, pl.ds(off, _LANES)]
                    ov = o_v[r, pl.ds(off, _LANES)]
                    return c1 + wv * ov, c2 + wv

                a1, a2 = lax.fori_loop(
                    0, cf, col_body, (a1, jnp.zeros((_LANES,), jnp.float32)))
                if ct:
                    wv = w_v[r, pl.ds(C - _LANES, _LANES)] * tailmask
                    ov = o_v[r, pl.ds(C - _LANES, _LANES)]
                    a1 = a1 + wv * ov
                    a2 = a2 + wv
                g2_v[ch * K + r, :] = a2
                return a1

            acc1 = lax.fori_loop(0, K, row_body, acc1)

        t1_v[:] = acc1
        pltpu.sync_copy(g2_v, g2_hbm.at[pl.ds(base, bpw)])
        pltpu.sync_copy(t1_v, t1_hbm.at[wid])

    return k(output, idx3, weights_p)


def _tc_combine(output, g2part, t1part, *, B, C):
    BLK = 256
    grid = (B // BLK,)

    def body(out_ref, g2_ref, t1_ref, L_ref):
        j = pl.program_id(0)
        x = out_ref[...]
        m = jnp.max(x, axis=1, keepdims=True)
        lse = m + jnp.log(jnp.sum(jnp.exp(x - m), axis=1, keepdims=True))
        g2 = jnp.sum(g2_ref[...], axis=1, keepdims=True)

        @pl.when(j == 0)
        def _():
            L_ref[...] = -jnp.sum(t1_ref[...], keepdims=True).reshape(1, 1) / B

        L_ref[...] += jnp.sum(lse * g2, keepdims=True).reshape(1, 1) / B

    L = pl.pallas_call(
        body,
        grid=grid,
        in_specs=[
            pl.BlockSpec((BLK, C), lambda j: (j, 0)),
            pl.BlockSpec((BLK, _LANES), lambda j: (j, 0)),
            pl.BlockSpec((_NW, _LANES), lambda j: (0, 0)),
        ],
        out_specs=pl.BlockSpec((1, 1), lambda j: (0, 0)),
        out_shape=jax.ShapeDtypeStruct((1, 1), jnp.float32),
    )(output, g2part, t1part)
    return L[0, 0]


def kernel(output, targets, indices, weights):
    B, C = output.shape
    CP = (C + 127) // 128 * 128
    weights_p = _tc_pad_rows(weights, CP=CP) if CP != C else weights
    bpw = B // _NW
    K = 16
    nchunk = bpw // K
    idx3 = indices.reshape(_NW, nchunk, K)
    g2part, t1part = _sc_gather_stats(
        output, idx3, weights_p, B=B, C=C, bpw=bpw, K=K, nchunk=nchunk)
    return _tc_combine(output, g2part, t1part, B=B, C=C)


# speedup vs baseline: 4.3474x; 1.6076x over previous
import functools

import jax
import jax.numpy as jnp
from jax import lax
from jax.experimental import pallas as pl
from jax.experimental.pallas import tpu as pltpu
from jax.experimental.pallas import tpu_sc as plsc

_NC = 2
_NS = 16
_NW = _NC * _NS
_LANES = 16


def _sc_head_stats(output, idx3, weights, *, B, CH, bpw, K, nchunk):
    cf = CH // _LANES

    mesh = plsc.VectorSubcoreMesh(core_axis_name="c", subcore_axis_name="s")

    @functools.partial(
        pl.kernel,
        mesh=mesh,
        out_type=[
            jax.ShapeDtypeStruct((B, _LANES), jnp.float32),
            jax.ShapeDtypeStruct((_NW, _LANES), jnp.float32),
        ],
        scratch_types=[
            pltpu.VMEM((nchunk, K), jnp.int32),
            pltpu.VMEM((K, CH), jnp.float32),
            pltpu.VMEM((K, CH), jnp.float32),
            pltpu.VMEM((bpw, _LANES), jnp.float32),
            pltpu.VMEM((_LANES,), jnp.float32),
        ],
    )
    def k(out_hbm, idx_hbm, w_hbm, g2_hbm, t1_hbm, idx_v, w_v, o_v, g2_v, t1_v):
        cid = lax.axis_index("c")
        sid = lax.axis_index("s")
        wid = sid * _NC + cid
        base = wid * bpw

        pltpu.sync_copy(idx_hbm.at[wid], idx_v)

        acc1 = jnp.zeros((_LANES,), jnp.float32)
        for ch in range(nchunk):
            pltpu.sync_copy(w_hbm.at[idx_v.at[ch], pl.ds(0, CH)], w_v)
            pltpu.sync_copy(out_hbm.at[pl.ds(base + ch * K, K), pl.ds(0, CH)], o_v)

            def row_body(r, a1):
                def col_body(j, carry):
                    c1, c2 = carry
                    off = pl.multiple_of(j * _LANES, _LANES)
                    wv = w_v[r, pl.ds(off, _LANES)]
                    ov = o_v[r, pl.ds(off, _LANES)]
                    return c1 + wv * ov, c2 + wv

                a1, a2 = lax.fori_loop(
                    0, cf, col_body, (a1, jnp.zeros((_LANES,), jnp.float32)))
                g2_v[ch * K + r, :] = a2
                return a1

            acc1 = lax.fori_loop(0, K, row_body, acc1)

        t1_v[:] = acc1
        pltpu.sync_copy(g2_v, g2_hbm.at[pl.ds(base, bpw)])
        pltpu.sync_copy(t1_v, t1_hbm.at[wid])

    return k(output, idx3, weights)


def _tc_lse_tail(output, idx2, weights, *, B, C, CH):
    BLK = 128
    CT = C - CH
    grid = (B // BLK,)

    def body(idx_ref, out_ref, w_hbm, lse_ref, s_ref, tail_ref, sem):
        j = pl.program_id(0)

        for r in range(BLK):
            pltpu.make_async_copy(
                w_hbm.at[pl.ds(idx_ref[0, 0, r], 1), pl.ds(CH, CT)],
                tail_ref.at[pl.ds(r, 1), :],
                sem,
            ).start()

        x = out_ref[...]
        m = jnp.max(x, axis=1, keepdims=True)
        lse = m + jnp.log(jnp.sum(jnp.exp(x - m), axis=1, keepdims=True))
        lse_ref[...] = lse

        for r in range(BLK):
            pltpu.make_async_copy(
                w_hbm.at[pl.ds(0, 1), pl.ds(CH, CT)],
                tail_ref.at[pl.ds(r, 1), :],
                sem,
            ).wait()

        tw = tail_ref[...]
        tout = out_ref[:, CH:C]
        tg2 = jnp.sum(tw, axis=1, keepdims=True)

        @pl.when(j == 0)
        def _():
            s_ref[...] = jnp.zeros((1, 1), jnp.float32)

        s_ref[...] += (jnp.sum(lse * tg2, keepdims=True).reshape(1, 1)
                       - jnp.sum(tw * tout, keepdims=True).reshape(1, 1))

    return pl.pallas_call(
        body,
        grid=grid,
        in_specs=[
            pl.BlockSpec((1, 1, BLK), lambda j: (j, 0, 0), memory_space=pltpu.SMEM),
            pl.BlockSpec((BLK, C), lambda j: (j, 0)),
            pl.BlockSpec(memory_space=pl.ANY),
        ],
        out_specs=[
            pl.BlockSpec((BLK, 1), lambda j: (j, 0)),
            pl.BlockSpec((1, 1), lambda j: (0, 0)),
        ],
        out_shape=[
            jax.ShapeDtypeStruct((B, 1), jnp.float32),
            jax.ShapeDtypeStruct((1, 1), jnp.float32),
        ],
        scratch_shapes=[
            pltpu.VMEM((BLK, CT), jnp.float32),
            pltpu.SemaphoreType.DMA,
        ],
    )(idx2, output, weights)


def _tc_combine(lse, g2part, t1part, s_tc, *, B):
    def body(lse_ref, g2_ref, t1_ref, s_ref, L_ref):
        g2 = jnp.sum(g2_ref[...], axis=1, keepdims=True)
        L_ref[...] = (
            jnp.sum(lse_ref[...] * g2, keepdims=True).reshape(1, 1)
            - jnp.sum(t1_ref[...], keepdims=True).reshape(1, 1)
            + s_ref[...]
        ) / B

    L = pl.pallas_call(
        body,
        out_shape=jax.ShapeDtypeStruct((1, 1), jnp.float32),
    )(lse, g2part, t1part, s_tc)
    return L[0, 0]


def kernel(output, targets, indices, weights):
    B, C = output.shape
    CH = C // 128 * 128
    bpw = B // _NW
    K = 16
    nchunk = bpw // K
    idx3 = indices.reshape(_NW, nchunk, K)
    idx2 = indices.reshape(B // 128, 1, 128)
    g2part, t1part = _sc_head_stats(
        output, idx3, weights, B=B, CH=CH, bpw=bpw, K=K, nchunk=nchunk)
    lse, s_tc = _tc_lse_tail(output, idx2, weights, B=B, C=C, CH=CH)
    return _tc_combine(lse, g2part, t1part, s_tc, B=B)
